# baseline (speedup 1.0000x reference)
import functools

import jax
import jax.numpy as jnp
from jax import lax
from jax.experimental import pallas as pl
from jax.experimental.pallas import tpu as pltpu
from jax.experimental.pallas import tpu_sc as plsc

NUM_TOKENS = 16384
HIDDEN = 2048
NUM_EXPERTS = 64
BLOCK_TOKENS = 2048

NC, NS = 2, 16
NW = NC * NS
ROWS_PER_W = NUM_TOKENS // NW


def _matmul_kernel(x_ref, wt_ref, scores_ref):
    scores_ref[...] = jnp.dot(x_ref[...], wt_ref[...],
                              preferred_element_type=jnp.float32)


def _tc_scores(x, Wt):
    grid = (NUM_TOKENS // BLOCK_TOKENS,)
    return pl.pallas_call(
        _matmul_kernel,
        grid=grid,
        in_specs=[
            pl.BlockSpec((BLOCK_TOKENS, HIDDEN), lambda i: (i, 0)),
            pl.BlockSpec((HIDDEN, NUM_EXPERTS), lambda i: (0, 0)),
        ],
        out_specs=pl.BlockSpec((BLOCK_TOKENS, NUM_EXPERTS), lambda i: (i, 0)),
        out_shape=jax.ShapeDtypeStruct((NUM_TOKENS, NUM_EXPERTS), jnp.float32),
    )(x, Wt)


@functools.partial(
    pl.kernel,
    out_type=[
        jax.ShapeDtypeStruct((NUM_TOKENS, 2), jnp.float32),
        jax.ShapeDtypeStruct((NUM_TOKENS, 2), jnp.int32),
    ],
    mesh=plsc.VectorSubcoreMesh(core_axis_name="c", subcore_axis_name="s"),
    compiler_params=pltpu.CompilerParams(needs_layout_passes=False,
                                         use_tc_tiling_on_sc=False),
    scratch_types=[
        pltpu.VMEM((ROWS_PER_W, NUM_EXPERTS), jnp.float32),
        pltpu.VMEM((ROWS_PER_W, 2), jnp.float32),
        pltpu.VMEM((ROWS_PER_W, 2), jnp.int32),
    ],
)
def _sc_top2(scores_hbm, vals_hbm, idx_hbm, sc_v, vals_v, idx_v):
    wid = lax.axis_index("s") * NC + lax.axis_index("c")
    base = wid * ROWS_PER_W
    pltpu.sync_copy(scores_hbm.at[pl.ds(base, ROWS_PER_W)], sc_v)

    iota = lax.iota(jnp.int32, 16)
    zeros = jnp.zeros((16,), jnp.int32)
    ones = jnp.full((16,), 1, jnp.int32)

    def group(g, carry):
        row = g * 16 + iota
        m1 = jnp.full((16,), -jnp.inf, jnp.float32)
        m2 = jnp.full((16,), -jnp.inf, jnp.float32)
        i1 = zeros
        i2 = zeros
        def col_step(_, st):
            m1, m2, i1, i2, e_v = st
            c = plsc.load_gather(sc_v, [row, e_v])
            gt1 = c > m1
            gt2 = c > m2
            m2n = jnp.where(gt1, m1, jnp.where(gt2, c, m2))
            i2n = jnp.where(gt1, i1, jnp.where(gt2, e_v, i2))
            m1n = jnp.where(gt1, c, m1)
            i1n = jnp.where(gt1, e_v, i1)
            return (m1n, m2n, i1n, i2n, e_v + ones)

        m1, m2, i1, i2, _ = lax.fori_loop(
            0, NUM_EXPERTS, col_step, (m1, m2, i1, i2, zeros))
        plsc.store_scatter(vals_v, [row, zeros], m1)
        plsc.store_scatter(vals_v, [row, ones], m2)
        plsc.store_scatter(idx_v, [row, zeros], i1)
        plsc.store_scatter(idx_v, [row, ones], i2)
        return carry

    lax.fori_loop(0, ROWS_PER_W // 16, group, 0)
    pltpu.sync_copy(vals_v, vals_hbm.at[pl.ds(base, ROWS_PER_W)])
    pltpu.sync_copy(idx_v, idx_hbm.at[pl.ds(base, ROWS_PER_W)])


@jax.jit
def kernel(x, W):
    scores = _tc_scores(x, W.T)
    vals, idx = _sc_top2(scores)
    return vals, idx
